# Initial kernel scaffold; baseline (speedup 1.0000x reference)
import functools

import jax
import jax.numpy as jnp
from jax import lax
from jax.experimental import pallas as pl
from jax.experimental.pallas import tpu as pltpu
from jax.experimental.pallas import tpu_sc as plsc

N = 10000
D = 128
NC = 2
NS = 16
NW = NC * NS
CH = 128
NCH = 80
EPT = NCH * CH
E_PAD = NW * EPT
ACC_ROWS = 10240
RPT = ACC_ROWS // NS
ZR = 160
K = 4
RB = 2048

_f32 = jnp.float32


def _agg(src_h, dst_h, tbl_h, s_h, c_h, acc, cnt, sidx, didx, rbufs, zbuf,
         onesb, cbuf, gsems, cid, sid, with_counts):
    for k in range(RPT // ZR):
        pltpu.sync_copy(zbuf, acc.at[pl.ds(sid * RPT + k * ZR, ZR), :])
    if with_counts:
        for k in range(RPT // CH):
            pltpu.sync_copy(zbuf.at[0, :], cnt.at[pl.ds(sid * RPT + k * CH, CH)])
    wid = cid * NS + sid
    pltpu.sync_copy(src_h.at[wid], sidx)
    pltpu.sync_copy(dst_h.at[wid], didx)
    plsc.subcore_barrier()

    for b in range(K):
        pltpu.async_copy(tbl_h.at[sidx.at[b]], rbufs[b], gsems[b])

    @pl.loop(0, NCH // K)
    def _grp(g):
        for b in range(K):
            j = g * K + b
            pltpu.make_async_copy(tbl_h.at[sidx.at[b]], rbufs[b], gsems[b]).wait()
            pltpu.sync_copy(rbufs[b], acc.at[didx.at[j]], add=True)
            if with_counts:
                pltpu.sync_copy(onesb, cnt.at[didx.at[j]], add=True)

            @pl.when(j + K < NCH)
            def _pf():
                pltpu.async_copy(tbl_h.at[sidx.at[j + K]], rbufs[b], gsems[b])

    plsc.subcore_barrier()
    for k in range(RPT // CH):
        pltpu.sync_copy(acc.at[pl.ds(sid * RPT + k * CH, CH), :], rbufs[0])
        pltpu.sync_copy(rbufs[0], s_h.at[cid, pl.ds(sid * RPT + k * CH, CH), :])
    if with_counts:
        pltpu.sync_copy(cnt.at[pl.ds(sid * RPT, RPT)], cbuf)
        pltpu.sync_copy(cbuf, c_h.at[cid, pl.ds(sid * RPT, RPT)])
    plsc.subcore_barrier()


def _init_consts(zbuf, onesb):
    @pl.loop(0, ZR * (D // 16))
    def _z(i):
        zbuf[i // (D // 16), pl.ds((i % (D // 16)) * 16, 16)] = (
            jnp.zeros((16,), _f32))

    if onesb is not None:
        @pl.loop(0, CH // 16)
        def _o(i):
            onesb[pl.ds(i * 16, 16)] = jnp.ones((16,), _f32)


def _sc_two_aggs(si, di, y1, su, du, y2):
    mesh = plsc.VectorSubcoreMesh(core_axis_name="c", subcore_axis_name="s")

    @functools.partial(
        pl.kernel,
        out_type=(
            jax.ShapeDtypeStruct((NC, ACC_ROWS, D), _f32),
            jax.ShapeDtypeStruct((NC, ACC_ROWS), _f32),
            jax.ShapeDtypeStruct((NC, ACC_ROWS, D), _f32),
            jax.ShapeDtypeStruct((NC, ACC_ROWS), _f32),
        ),
        mesh=mesh,
        scratch_types=[
            pltpu.VMEM_SHARED((ACC_ROWS, D), _f32),
            pltpu.VMEM_SHARED((ACC_ROWS,), _f32),
            pltpu.VMEM((NCH, CH), jnp.int32),
            pltpu.VMEM((NCH, CH), jnp.int32),
            pltpu.VMEM((CH, D), _f32),
            pltpu.VMEM((CH, D), _f32),
            pltpu.VMEM((CH, D), _f32),
            pltpu.VMEM((CH, D), _f32),
            pltpu.VMEM((ZR, D), _f32),
            pltpu.VMEM((CH,), _f32),
            pltpu.VMEM((RPT,), _f32),
            pltpu.SemaphoreType.DMA,
            pltpu.SemaphoreType.DMA,
            pltpu.SemaphoreType.DMA,
            pltpu.SemaphoreType.DMA,
        ],
    )
    def body(si_h, di_h, y1_h, su_h, du_h, y2_h, s1_h, c1_h, s2_h, c2_h,
             acc, cnt, sidx, didx, r0, r1, r2, r3, zbuf, onesb, cbuf,
             g0, g1, g2, g3):
        cid = lax.axis_index("c")
        sid = lax.axis_index("s")
        rbufs = (r0, r1, r2, r3)
        gsems = (g0, g1, g2, g3)
        _init_consts(zbuf, onesb)
        _agg(si_h, di_h, y1_h, s1_h, c1_h, acc, cnt, sidx, didx, rbufs,
             zbuf, onesb, cbuf, gsems, cid, sid, True)
        _agg(su_h, du_h, y2_h, s2_h, c2_h, acc, cnt, sidx, didx, rbufs,
             zbuf, onesb, cbuf, gsems, cid, sid, True)

    return body(si, di, y1, su, du, y2)


def _sc_one_agg(su, du, y3):
    mesh = plsc.VectorSubcoreMesh(core_axis_name="c", subcore_axis_name="s")

    @functools.partial(
        pl.kernel,
        out_type=jax.ShapeDtypeStruct((NC, ACC_ROWS, D), _f32),
        mesh=mesh,
        scratch_types=[
            pltpu.VMEM_SHARED((ACC_ROWS, D), _f32),
            pltpu.VMEM((NCH, CH), jnp.int32),
            pltpu.VMEM((NCH, CH), jnp.int32),
            pltpu.VMEM((CH, D), _f32),
            pltpu.VMEM((CH, D), _f32),
            pltpu.VMEM((CH, D), _f32),
            pltpu.VMEM((CH, D), _f32),
            pltpu.VMEM((ZR, D), _f32),
            pltpu.SemaphoreType.DMA,
            pltpu.SemaphoreType.DMA,
            pltpu.SemaphoreType.DMA,
            pltpu.SemaphoreType.DMA,
        ],
    )
    def body(su_h, du_h, y3_h, s3_h, acc, sidx, didx, r0, r1, r2, r3,
             zbuf, g0, g1, g2, g3):
        cid = lax.axis_index("c")
        sid = lax.axis_index("s")
        rbufs = (r0, r1, r2, r3)
        gsems = (g0, g1, g2, g3)
        _init_consts(zbuf, None)
        _agg(su_h, du_h, y3_h, s3_h, None, acc, None, sidx, didx, rbufs,
             zbuf, None, None, gsems, cid, sid, False)

    return body(su, du, y3)


def _row_spec(r=RB):
    return pl.BlockSpec((r, D), lambda i: (i, 0))


def _full_spec(shape):
    nd = len(shape)
    return pl.BlockSpec(shape, lambda i: (0,) * nd)


def _part_spec():
    return pl.BlockSpec((NC, RB, D), lambda i: (0, i, 0))


def _cnt_spec():
    return pl.BlockSpec((NC, RB), lambda i: (0, i))


_GRID = (ACC_ROWS // RB,)


def _tc1(xi, xu, w1l, w2l, w1r, w2r, b1, b2):
    def body(xi_r, xu_r, w1l_r, w2l_r, w1r_r, w2r_r, b1_r, b2_r,
             y1_r, y2_r, r1_r, r2_r):
        a = xi_r[...]
        u = xu_r[...]
        y1_r[...] = jnp.dot(a, w1l_r[...], preferred_element_type=_f32)
        y2_r[...] = jnp.dot(a, w2l_r[...], preferred_element_type=_f32)
        r1_r[...] = jnp.dot(a, w1r_r[...], preferred_element_type=_f32) + b1_r[...]
        r2_r[...] = jnp.dot(u, w2r_r[...], preferred_element_type=_f32) + b2_r[...]

    o = jax.ShapeDtypeStruct((N, D), _f32)
    return pl.pallas_call(
        body,
        grid=_GRID,
        in_specs=[_row_spec(), _row_spec(), _full_spec((D, D)), _full_spec((D, D)),
                  _full_spec((D, D)), _full_spec((D, D)), _full_spec((1, D)),
                  _full_spec((1, D))],
        out_specs=[_row_spec()] * 4,
        out_shape=[o, o, o, o],
    )(xi, xu, w1l, w2l, w1r, w2r, b1, b2)


def _tc2(s1p, c1p, r1, s2p, c2p, r2, w3l, w3r, b3):
    def body(s1_r, c1_r, r1_r, s2_r, c2_r, r2_r, w3l_r, w3r_r, b3_r,
             y3_r, r3_r):
        c1 = jnp.maximum(c1_r[...][0] + c1_r[...][1], 1.0)
        s1 = s1_r[...][0] + s1_r[...][1]
        item_x = jnp.maximum(s1 / c1[:, None] + r1_r[...], 0.0)
        y3_r[...] = jnp.dot(item_x, w3l_r[...], preferred_element_type=_f32)
        c2 = jnp.maximum(c2_r[...][0] + c2_r[...][1], 1.0)
        s2 = s2_r[...][0] + s2_r[...][1]
        user_x = jnp.maximum(s2 / c2[:, None] + r2_r[...], 0.0)
        r3_r[...] = jnp.dot(user_x, w3r_r[...], preferred_element_type=_f32) + b3_r[...]

    o = jax.ShapeDtypeStruct((N, D), _f32)
    return pl.pallas_call(
        body,
        grid=_GRID,
        in_specs=[_part_spec(), _cnt_spec(), _row_spec(), _part_spec(),
                  _cnt_spec(), _row_spec(), _full_spec((D, D)),
                  _full_spec((D, D)), _full_spec((1, D))],
        out_specs=[_row_spec()] * 2,
        out_shape=[o, o],
    )(s1p, c1p, r1, s2p, c2p, r2, w3l, w3r, b3)


def _tc3(s3p, c2p, r3, wlin, blin):
    def body(s3_r, c2_r, r3_r, wl_r, bl_r, out_r):
        c2 = jnp.maximum(c2_r[...][0] + c2_r[...][1], 1.0)
        s3 = s3_r[...][0] + s3_r[...][1]
        u3 = jnp.maximum(s3 / c2[:, None] + r3_r[...], 0.0)
        out_r[...] = jnp.dot(u3, wl_r[...], preferred_element_type=_f32) + bl_r[...]

    return pl.pallas_call(
        body,
        grid=_GRID,
        in_specs=[_part_spec(), _cnt_spec(), _row_spec(), _full_spec((D, D)),
                  _full_spec((1, D))],
        out_specs=_row_spec(),
        out_shape=jax.ShapeDtypeStruct((N, D), _f32),
    )(s3p, c2p, r3, wlin, blin)


def _prep_edges(edge_index):
    src = edge_index[0].astype(jnp.int32)
    dst = edge_index[1].astype(jnp.int32)
    npad = E_PAD - src.shape[0]
    ar = jnp.arange(npad, dtype=jnp.int32)
    pad_src = ar % N
    pad_dst = N + ar % (ACC_ROWS - N)
    src = jnp.concatenate([src, pad_src]).reshape(NW, NCH, CH)
    dst = jnp.concatenate([dst, pad_dst]).reshape(NW, NCH, CH)
    return src, dst


def kernel(x_item, x_user, edge_index_ii, edge_index_iu, W1l, b1l, W1r,
           W2l, b2l, W2r, W3l, b3l, W3r, Wlin, blin):
    si, di = _prep_edges(edge_index_ii)
    su, du = _prep_edges(edge_index_iu)
    b1 = b1l.reshape(1, D)
    b2 = b2l.reshape(1, D)
    b3 = b3l.reshape(1, D)
    bl = blin.reshape(1, D)

    y1, y2, r1, r2 = _tc1(x_item, x_user, W1l.T, W2l.T, W1r.T, W2r.T, b1, b2)
    s1p, c1p, s2p, c2p = _sc_two_aggs(si, di, y1, su, du, y2)
    y3, r3 = _tc2(s1p, c1p, r1, s2p, c2p, r2, W3l.T, W3r.T, b3)
    s3p = _sc_one_agg(su, du, y3)
    return _tc3(s3p, c2p, r3, Wlin.T, bl)


# speedup vs baseline: 12.0042x; 12.0042x over previous
import functools

import jax
import jax.numpy as jnp
from jax import lax
from jax.experimental import pallas as pl
from jax.experimental.pallas import tpu as pltpu
from jax.experimental.pallas import tpu_sc as plsc

N = 10000
D = 128
NC = 2
NS = 16
NW = NC * NS
CH = 128
NCH = 80
HNCH = NCH // 2
EPT = NCH * CH
E_PAD = NW * EPT
ACC_ROWS = 10112
RPT = ACC_ROWS // NS
K = 2
RB = 2048

_f32 = jnp.float32


def _zero_rbuf(rb):
    @pl.loop(0, CH * (D // 16))
    def _z(i):
        rb[i // (D // 16), pl.ds((i % (D // 16)) * 16, 16)] = (
            jnp.zeros((16,), _f32))


_SLABS = [(o, min(CH, RPT - o)) for o in range(0, RPT, CH)]


def _agg(src_h, dst_h, tbl_h, s_h, c_h, acc, cnt, sidx, didx, rbufs,
         onesb, cbuf, gsems, cid, sid, with_counts):
    _zero_rbuf(rbufs[0])
    for off, n in _SLABS:
        pltpu.sync_copy(rbufs[0].at[pl.ds(0, n), :],
                        acc.at[pl.ds(sid * RPT + off, n), :])
    if with_counts:
        for off, n in _SLABS:
            pltpu.sync_copy(rbufs[0].at[0, pl.ds(0, n)],
                            cnt.at[pl.ds(sid * RPT + off, n)])
    plsc.subcore_barrier()

    wid = cid * NS + sid
    for h in range(NCH // HNCH):
        pltpu.sync_copy(src_h.at[wid, pl.ds(h * HNCH, HNCH)], sidx)
        pltpu.sync_copy(dst_h.at[wid, pl.ds(h * HNCH, HNCH)], didx)
        for b in range(K):
            pltpu.async_copy(tbl_h.at[sidx.at[b]], rbufs[b], gsems[b])

        @pl.loop(0, HNCH // K)
        def _grp(g):
            for b in range(K):
                j = g * K + b
                pltpu.make_async_copy(
                    tbl_h.at[sidx.at[b]], rbufs[b], gsems[b]).wait()
                pltpu.sync_copy(rbufs[b], acc.at[didx.at[j]], add=True)
                if with_counts:
                    pltpu.sync_copy(onesb, cnt.at[didx.at[j]], add=True)

                @pl.when(j + K < HNCH)
                def _pf():
                    pltpu.async_copy(tbl_h.at[sidx.at[j + K]], rbufs[b],
                                     gsems[b])

    plsc.subcore_barrier()
    for off, n in _SLABS:
        pltpu.sync_copy(acc.at[pl.ds(sid * RPT + off, n), :],
                        rbufs[0].at[pl.ds(0, n), :])
        pltpu.sync_copy(rbufs[0].at[pl.ds(0, n), :],
                        s_h.at[cid, pl.ds(sid * RPT + off, n), :])
    if with_counts:
        pltpu.sync_copy(cnt.at[pl.ds(sid * RPT, RPT)], cbuf)
        pltpu.sync_copy(cbuf, c_h.at[pl.ds(cid * ACC_ROWS + sid * RPT, RPT)])
    plsc.subcore_barrier()


def _init_consts(onesb):
    if onesb is not None:
        @pl.loop(0, CH // 16)
        def _o(i):
            onesb[pl.ds(i * 16, 16)] = jnp.ones((16,), _f32)


def _sc_two_aggs(si, di, y1, su, du, y2):
    mesh = plsc.VectorSubcoreMesh(core_axis_name="c", subcore_axis_name="s")

    @functools.partial(
        pl.kernel,
        out_type=(
            jax.ShapeDtypeStruct((NC, ACC_ROWS, D), _f32),
            jax.ShapeDtypeStruct((NC * ACC_ROWS,), _f32),
            jax.ShapeDtypeStruct((NC, ACC_ROWS, D), _f32),
            jax.ShapeDtypeStruct((NC * ACC_ROWS,), _f32),
        ),
        mesh=mesh,
        scratch_types=[
            pltpu.VMEM_SHARED((ACC_ROWS, D), _f32),
            pltpu.VMEM_SHARED((ACC_ROWS,), _f32),
            pltpu.VMEM((HNCH, CH), jnp.int32),
            pltpu.VMEM((HNCH, CH), jnp.int32),
            pltpu.VMEM((CH, D), _f32),
            pltpu.VMEM((CH, D), _f32),
            pltpu.VMEM((CH,), _f32),
            pltpu.VMEM((RPT,), _f32),
            pltpu.SemaphoreType.DMA,
            pltpu.SemaphoreType.DMA,
        ],
    )
    def body(si_h, di_h, y1_h, su_h, du_h, y2_h, s1_h, c1_h, s2_h, c2_h,
             acc, cnt, sidx, didx, r0, r1, onesb, cbuf, g0, g1):
        cid = lax.axis_index("c")
        sid = lax.axis_index("s")
        rbufs = (r0, r1)
        gsems = (g0, g1)
        _init_consts(onesb)
        _agg(si_h, di_h, y1_h, s1_h, c1_h, acc, cnt, sidx, didx, rbufs,
             onesb, cbuf, gsems, cid, sid, True)
        _agg(su_h, du_h, y2_h, s2_h, c2_h, acc, cnt, sidx, didx, rbufs,
             onesb, cbuf, gsems, cid, sid, True)

    return body(si, di, y1, su, du, y2)


def _sc_one_agg(su, du, y3):
    mesh = plsc.VectorSubcoreMesh(core_axis_name="c", subcore_axis_name="s")

    @functools.partial(
        pl.kernel,
        out_type=jax.ShapeDtypeStruct((NC, ACC_ROWS, D), _f32),
        mesh=mesh,
        scratch_types=[
            pltpu.VMEM_SHARED((ACC_ROWS, D), _f32),
            pltpu.VMEM((HNCH, CH), jnp.int32),
            pltpu.VMEM((HNCH, CH), jnp.int32),
            pltpu.VMEM((CH, D), _f32),
            pltpu.VMEM((CH, D), _f32),
            pltpu.SemaphoreType.DMA,
            pltpu.SemaphoreType.DMA,
        ],
    )
    def body(su_h, du_h, y3_h, s3_h, acc, sidx, didx, r0, r1, g0, g1):
        cid = lax.axis_index("c")
        sid = lax.axis_index("s")
        rbufs = (r0, r1)
        gsems = (g0, g1)
        _agg(su_h, du_h, y3_h, s3_h, None, acc, None, sidx, didx, rbufs,
             None, None, gsems, cid, sid, False)

    return body(su, du, y3)


def _row_spec(r=RB):
    return pl.BlockSpec((r, D), lambda i: (i, 0))


def _full_spec(shape):
    nd = len(shape)
    return pl.BlockSpec(shape, lambda i: (0,) * nd)


def _part_spec():
    return pl.BlockSpec((NC, RB, D), lambda i: (0, i, 0))


def _cnt_spec():
    return pl.BlockSpec((NC, RB), lambda i: (0, i))


_GRID = ((N + RB - 1) // RB,)


def _tc1(xi, xu, w1l, w2l, w1r, w2r, b1, b2):
    def body(xi_r, xu_r, w1l_r, w2l_r, w1r_r, w2r_r, b1_r, b2_r,
             y1_r, y2_r, r1_r, r2_r):
        a = xi_r[...]
        u = xu_r[...]
        y1_r[...] = jnp.dot(a, w1l_r[...], preferred_element_type=_f32)
        y2_r[...] = jnp.dot(a, w2l_r[...], preferred_element_type=_f32)
        r1_r[...] = jnp.dot(a, w1r_r[...], preferred_element_type=_f32) + b1_r[...]
        r2_r[...] = jnp.dot(u, w2r_r[...], preferred_element_type=_f32) + b2_r[...]

    o = jax.ShapeDtypeStruct((N, D), _f32)
    return pl.pallas_call(
        body,
        grid=_GRID,
        in_specs=[_row_spec(), _row_spec(), _full_spec((D, D)), _full_spec((D, D)),
                  _full_spec((D, D)), _full_spec((D, D)), _full_spec((1, D)),
                  _full_spec((1, D))],
        out_specs=[_row_spec()] * 4,
        out_shape=[o, o, o, o],
    )(xi, xu, w1l, w2l, w1r, w2r, b1, b2)


def _tc2(s1p, c1p, r1, s2p, c2p, r2, w3l, w3r, b3):
    def body(s1_r, c1_r, r1_r, s2_r, c2_r, r2_r, w3l_r, w3r_r, b3_r,
             y3_r, r3_r):
        c1 = jnp.maximum(c1_r[...][0] + c1_r[...][1], 1.0)
        s1 = s1_r[...][0] + s1_r[...][1]
        item_x = jnp.maximum(s1 / c1[:, None] + r1_r[...], 0.0)
        y3_r[...] = jnp.dot(item_x, w3l_r[...], preferred_element_type=_f32)
        c2 = jnp.maximum(c2_r[...][0] + c2_r[...][1], 1.0)
        s2 = s2_r[...][0] + s2_r[...][1]
        user_x = jnp.maximum(s2 / c2[:, None] + r2_r[...], 0.0)
        r3_r[...] = jnp.dot(user_x, w3r_r[...], preferred_element_type=_f32) + b3_r[...]

    o = jax.ShapeDtypeStruct((N, D), _f32)
    return pl.pallas_call(
        body,
        grid=_GRID,
        in_specs=[_part_spec(), _cnt_spec(), _row_spec(), _part_spec(),
                  _cnt_spec(), _row_spec(), _full_spec((D, D)),
                  _full_spec((D, D)), _full_spec((1, D))],
        out_specs=[_row_spec()] * 2,
        out_shape=[o, o],
    )(s1p, c1p, r1, s2p, c2p, r2, w3l, w3r, b3)


def _tc3(s3p, c2p, r3, wlin, blin):
    def body(s3_r, c2_r, r3_r, wl_r, bl_r, out_r):
        c2 = jnp.maximum(c2_r[...][0] + c2_r[...][1], 1.0)
        s3 = s3_r[...][0] + s3_r[...][1]
        u3 = jnp.maximum(s3 / c2[:, None] + r3_r[...], 0.0)
        out_r[...] = jnp.dot(u3, wl_r[...], preferred_element_type=_f32) + bl_r[...]

    return pl.pallas_call(
        body,
        grid=_GRID,
        in_specs=[_part_spec(), _cnt_spec(), _row_spec(), _full_spec((D, D)),
                  _full_spec((1, D))],
        out_specs=_row_spec(),
        out_shape=jax.ShapeDtypeStruct((N, D), _f32),
    )(s3p, c2p, r3, wlin, blin)


def _prep_edges(edge_index):
    src = edge_index[0].astype(jnp.int32)
    dst = edge_index[1].astype(jnp.int32)
    npad = E_PAD - src.shape[0]
    ar = jnp.arange(npad, dtype=jnp.int32)
    pad_src = ar % N
    pad_dst = N + ar % (ACC_ROWS - N)
    src = jnp.concatenate([src, pad_src]).reshape(NW, NCH, CH)
    dst = jnp.concatenate([dst, pad_dst]).reshape(NW, NCH, CH)
    return src, dst


def kernel(x_item, x_user, edge_index_ii, edge_index_iu, W1l, b1l, W1r,
           W2l, b2l, W2r, W3l, b3l, W3r, Wlin, blin):
    si, di = _prep_edges(edge_index_ii)
    su, du = _prep_edges(edge_index_iu)
    b1 = b1l.reshape(1, D)
    b2 = b2l.reshape(1, D)
    b3 = b3l.reshape(1, D)
    bl = blin.reshape(1, D)

    y1, y2, r1, r2 = _tc1(x_item, x_user, W1l.T, W2l.T, W1r.T, W2r.T, b1, b2)
    s1p, c1p, s2p, c2p = _sc_two_aggs(si, di, y1, su, du, y2)
    c1p = c1p.reshape(NC, ACC_ROWS)
    c2p = c2p.reshape(NC, ACC_ROWS)
    y3, r3 = _tc2(s1p, c1p, r1, s2p, c2p, r2, W3l.T, W3r.T, b3)
    s3p = _sc_one_agg(su, du, y3)
    return _tc3(s3p, c2p, r3, Wlin.T, bl)


# speedup vs baseline: 12.7697x; 1.0638x over previous
import functools

import jax
import jax.numpy as jnp
from jax import lax
from jax.experimental import pallas as pl
from jax.experimental.pallas import tpu as pltpu
from jax.experimental.pallas import tpu_sc as plsc

N = 10000
D = 128
NC = 2
NS = 16
NW = NC * NS
CH = 128
SLAB = 40
EPT = 20480
E_PAD = NS * EPT
IDXROWS = E_PAD // CH
ACC_ROWS = 10112
RPT = ACC_ROWS // NS
K = 2
RB = 2048

_f32 = jnp.float32

_SLABS = [(o, min(CH, RPT - o)) for o in range(0, RPT, CH)]


def _zero_rbuf(rb):
    @pl.loop(0, CH * (D // 16))
    def _z(i):
        rb[i // (D // 16), pl.ds((i % (D // 16)) * 16, 16)] = (
            jnp.zeros((16,), _f32))


def _agg(src_h, dst_h, tbl_h, s_out, c_out, acc, cnt, sidx, didx, rbufs,
         onesb, cbuf, gsems, sid, row0, nstages, with_counts):
    _zero_rbuf(rbufs[0])
    for off, n in _SLABS:
        pltpu.sync_copy(rbufs[0].at[pl.ds(0, n), :],
                        acc.at[pl.ds(sid * RPT + off, n), :])
    if with_counts:
        for off, n in _SLABS:
            pltpu.sync_copy(rbufs[0].at[0, pl.ds(0, n)],
                            cnt.at[pl.ds(sid * RPT + off, n)])
    plsc.subcore_barrier()

    for st in range(nstages):
        r0 = row0 + st * SLAB
        pltpu.sync_copy(src_h.at[pl.ds(r0, SLAB), :], sidx)
        pltpu.sync_copy(dst_h.at[pl.ds(r0, SLAB), :], didx)
        for b in range(K):
            pltpu.async_copy(tbl_h.at[sidx.at[b]], rbufs[b], gsems[b])

        @pl.loop(0, SLAB // K)
        def _grp(g):
            for b in range(K):
                j = g * K + b
                pltpu.make_async_copy(
                    tbl_h.at[sidx.at[b]], rbufs[b], gsems[b]).wait()
                pltpu.sync_copy(rbufs[b], acc.at[didx.at[j]], add=True)
                if with_counts:
                    pltpu.sync_copy(onesb, cnt.at[didx.at[j]], add=True)

                @pl.when(j + K < SLAB)
                def _pf():
                    pltpu.async_copy(tbl_h.at[sidx.at[j + K]], rbufs[b],
                                     gsems[b])

    plsc.subcore_barrier()
    for off, n in _SLABS:
        pltpu.sync_copy(acc.at[pl.ds(sid * RPT + off, n), :],
                        rbufs[0].at[pl.ds(0, n), :])
        pltpu.sync_copy(rbufs[0].at[pl.ds(0, n), :],
                        s_out.at[pl.ds(sid * RPT + off, n), :])
    if with_counts:
        pltpu.sync_copy(cnt.at[pl.ds(sid * RPT, RPT)], cbuf)
        pltpu.sync_copy(cbuf, c_out.at[pl.ds(sid * RPT, RPT)])
    plsc.subcore_barrier()


def _init_ones(onesb):
    @pl.loop(0, CH // 16)
    def _o(i):
        onesb[pl.ds(i * 16, 16)] = jnp.ones((16,), _f32)


_SC_SCRATCH = [
    pltpu.VMEM_SHARED((ACC_ROWS, D), _f32),
    pltpu.VMEM_SHARED((ACC_ROWS,), _f32),
    pltpu.VMEM((SLAB, CH), jnp.int32),
    pltpu.VMEM((SLAB, CH), jnp.int32),
    pltpu.VMEM((CH, D), _f32),
    pltpu.VMEM((CH, D), _f32),
    pltpu.VMEM((CH,), _f32),
    pltpu.VMEM((RPT,), _f32),
    pltpu.SemaphoreType.DMA,
    pltpu.SemaphoreType.DMA,
]


def _mesh():
    return plsc.VectorSubcoreMesh(core_axis_name="c", subcore_axis_name="s")


def _sc_two_aggs(si, di, y1, su, du, y2):
    @functools.partial(
        pl.kernel,
        out_type=(
            jax.ShapeDtypeStruct((ACC_ROWS, D), _f32),
            jax.ShapeDtypeStruct((ACC_ROWS,), _f32),
            jax.ShapeDtypeStruct((ACC_ROWS, D), _f32),
            jax.ShapeDtypeStruct((ACC_ROWS,), _f32),
        ),
        mesh=_mesh(),
        scratch_types=_SC_SCRATCH,
    )
    def body(si_h, di_h, y1_h, su_h, du_h, y2_h, s1_h, c1_h, s2_h, c2_h,
             acc, cnt, sidx, didx, r0, r1, onesb, cbuf, g0, g1):
        cid = lax.axis_index("c")
        sid = lax.axis_index("s")
        rbufs = (r0, r1)
        gsems = (g0, g1)
        _init_ones(onesb)
        row0 = sid * (EPT // CH)

        @pl.when(cid == 0)
        def _ii():
            _agg(si_h, di_h, y1_h, s1_h, c1_h, acc, cnt, sidx, didx, rbufs,
                 onesb, cbuf, gsems, sid, row0, EPT // CH // SLAB, True)

        @pl.when(cid == 1)
        def _iu():
            _agg(su_h, du_h, y2_h, s2_h, c2_h, acc, cnt, sidx, didx, rbufs,
                 onesb, cbuf, gsems, sid, row0, EPT // CH // SLAB, True)

    return body(si, di, y1, su, du, y2)


def _sc_one_agg(su, du, y3):
    @functools.partial(
        pl.kernel,
        out_type=jax.ShapeDtypeStruct((NC, ACC_ROWS, D), _f32),
        mesh=_mesh(),
        scratch_types=[
            pltpu.VMEM_SHARED((ACC_ROWS, D), _f32),
            pltpu.VMEM((SLAB, CH), jnp.int32),
            pltpu.VMEM((SLAB, CH), jnp.int32),
            pltpu.VMEM((CH, D), _f32),
            pltpu.VMEM((CH, D), _f32),
            pltpu.SemaphoreType.DMA,
            pltpu.SemaphoreType.DMA,
        ],
    )
    def body(su_h, du_h, y3_h, s3_h, acc, sidx, didx, r0, r1, g0, g1):
        cid = lax.axis_index("c")
        sid = lax.axis_index("s")
        wid = cid * NS + sid
        row0 = wid * (EPT // NC // CH)
        _agg(su_h, du_h, y3_h, s3_h.at[cid], None, acc, None, sidx, didx,
             (r0, r1), None, None, (g0, g1), sid, row0,
             EPT // NC // CH // SLAB, False)

    return body(su, du, y3)


def _row_spec():
    return pl.BlockSpec((RB, D), lambda i: (i, 0))


def _full_spec(shape):
    nd = len(shape)
    return pl.BlockSpec(shape, lambda i: (0,) * nd)


def _acc_spec():
    return pl.BlockSpec((RB, D), lambda i: (i, 0))


def _cnt_spec():
    return pl.BlockSpec((1, RB), lambda i: (0, i))


_GRID = ((N + RB - 1) // RB,)


def _tc_pre(xi, w1l, w2l):
    def body(xi_r, w1l_r, w2l_r, y1_r, y2_r):
        a = xi_r[...]
        y1_r[...] = jnp.dot(a, w1l_r[...], preferred_element_type=_f32)
        y2_r[...] = jnp.dot(a, w2l_r[...], preferred_element_type=_f32)

    o = jax.ShapeDtypeStruct((N, D), _f32)
    return pl.pallas_call(
        body,
        grid=_GRID,
        in_specs=[_row_spec(), _full_spec((D, D)), _full_spec((D, D))],
        out_specs=[_row_spec()] * 2,
        out_shape=[o, o],
    )(xi, w1l, w2l)


def _tc_roots(xi, xu, w1r, w2r, b1, b2):
    def body(xi_r, xu_r, w1r_r, w2r_r, b1_r, b2_r, r1_r, r2_r):
        r1_r[...] = jnp.dot(xi_r[...], w1r_r[...],
                            preferred_element_type=_f32) + b1_r[...]
        r2_r[...] = jnp.dot(xu_r[...], w2r_r[...],
                            preferred_element_type=_f32) + b2_r[...]

    o = jax.ShapeDtypeStruct((N, D), _f32)
    return pl.pallas_call(
        body,
        grid=_GRID,
        in_specs=[_row_spec(), _row_spec(), _full_spec((D, D)),
                  _full_spec((D, D)), _full_spec((1, D)), _full_spec((1, D))],
        out_specs=[_row_spec()] * 2,
        out_shape=[o, o],
    )(xi, xu, w1r, w2r, b1, b2)


def _tc_mid_y3(s1, c1, r1, w3l):
    def body(s1_r, c1_r, r1_r, w3l_r, y3_r):
        c1 = jnp.maximum(c1_r[...][0], 1.0)
        item_x = jnp.maximum(s1_r[...] / c1[:, None] + r1_r[...], 0.0)
        y3_r[...] = jnp.dot(item_x, w3l_r[...], preferred_element_type=_f32)

    return pl.pallas_call(
        body,
        grid=_GRID,
        in_specs=[_acc_spec(), _cnt_spec(), _row_spec(), _full_spec((D, D))],
        out_specs=_row_spec(),
        out_shape=jax.ShapeDtypeStruct((N, D), _f32),
    )(s1, c1, r1, w3l)


def _tc_mid_r3(s2, c2, r2, w3r, b3):
    def body(s2_r, c2_r, r2_r, w3r_r, b3_r, r3_r):
        c2 = jnp.maximum(c2_r[...][0], 1.0)
        user_x = jnp.maximum(s2_r[...] / c2[:, None] + r2_r[...], 0.0)
        r3_r[...] = jnp.dot(user_x, w3r_r[...],
                            preferred_element_type=_f32) + b3_r[...]

    return pl.pallas_call(
        body,
        grid=_GRID,
        in_specs=[_acc_spec(), _cnt_spec(), _row_spec(), _full_spec((D, D)),
                  _full_spec((1, D))],
        out_specs=_row_spec(),
        out_shape=jax.ShapeDtypeStruct((N, D), _f32),
    )(s2, c2, r2, w3r, b3)


def _tc_final(s3p, c2, r3, wlin, blin):
    def body(s3_r, c2_r, r3_r, wl_r, bl_r, out_r):
        cc = jnp.maximum(c2_r[...][0], 1.0)
        s3 = s3_r[...][0] + s3_r[...][1]
        u3 = jnp.maximum(s3 / cc[:, None] + r3_r[...], 0.0)
        out_r[...] = jnp.dot(u3, wl_r[...], preferred_element_type=_f32) + bl_r[...]

    return pl.pallas_call(
        body,
        grid=_GRID,
        in_specs=[pl.BlockSpec((NC, RB, D), lambda i: (0, i, 0)), _cnt_spec(),
                  _row_spec(), _full_spec((D, D)), _full_spec((1, D))],
        out_specs=_row_spec(),
        out_shape=jax.ShapeDtypeStruct((N, D), _f32),
    )(s3p, c2, r3, wlin, blin)


def _prep_edges(edge_index):
    src = edge_index[0].astype(jnp.int32)
    dst = edge_index[1].astype(jnp.int32)
    npad = E_PAD - src.shape[0]
    ar = jnp.arange(npad, dtype=jnp.int32)
    pad_src = ar % N
    pad_dst = N + ar % (ACC_ROWS - N)
    src = jnp.concatenate([src, pad_src]).reshape(IDXROWS, CH)
    dst = jnp.concatenate([dst, pad_dst]).reshape(IDXROWS, CH)
    return src, dst


def kernel(x_item, x_user, edge_index_ii, edge_index_iu, W1l, b1l, W1r,
           W2l, b2l, W2r, W3l, b3l, W3r, Wlin, blin):
    si, di = _prep_edges(edge_index_ii)
    su, du = _prep_edges(edge_index_iu)
    b1 = b1l.reshape(1, D)
    b2 = b2l.reshape(1, D)
    b3 = b3l.reshape(1, D)
    bl = blin.reshape(1, D)

    y1, y2 = _tc_pre(x_item, W1l.T, W2l.T)
    s1, c1, s2, c2 = _sc_two_aggs(si, di, y1, su, du, y2)
    c1 = c1.reshape(1, ACC_ROWS)
    c2 = c2.reshape(1, ACC_ROWS)
    r1, r2 = _tc_roots(x_item, x_user, W1r.T, W2r.T, b1, b2)
    y3 = _tc_mid_y3(s1, c1, r1, W3l.T)
    s3p = _sc_one_agg(su, du, y3)
    r3 = _tc_mid_r3(s2, c2, r2, W3r.T, b3)
    return _tc_final(s3p, c2, r3, Wlin.T, bl)
